# Initial kernel scaffold; baseline (speedup 1.0000x reference)
import functools

import jax
import jax.numpy as jnp
from jax import lax
from jax.experimental import pallas as pl
from jax.experimental.pallas import tpu as pltpu
from jax.experimental.pallas import tpu_sc as plsc

NN = 10000
NE = 320000
NC = 2
NS = 16
NW = NC * NS
CHUNK = 128
K = 80
NE_PAD = NW * K * CHUNK
ACC_ROWS = 10240
RPS = ACC_ROWS // NS
W1A = 80
W2A = 16

_mesh = plsc.VectorSubcoreMesh(core_axis_name="c", subcore_axis_name="s")


def _make_sc_segsum(width):
    @functools.partial(
        pl.kernel,
        mesh=_mesh,
        out_type=jax.ShapeDtypeStruct((NC, ACC_ROWS, width), jnp.float32),
        scratch_types=[
            pltpu.VMEM((K, CHUNK), jnp.int32),
            pltpu.VMEM((K, CHUNK), jnp.int32),
            pltpu.VMEM((CHUNK, width), jnp.float32),
            pltpu.VMEM_SHARED((ACC_ROWS, width), jnp.float32),
        ],
    )
    def segsum(table_hbm, src_hbm, dst_hbm, zero_hbm, out_hbm,
               idx_s, idx_d, rows, acc):
        c = lax.axis_index("c")
        s = lax.axis_index("s")
        wid = c * NS + s
        base = s * RPS
        pltpu.sync_copy(zero_hbm.at[pl.ds(base, RPS)], acc.at[pl.ds(base, RPS)])
        pltpu.sync_copy(src_hbm.at[wid], idx_s)
        pltpu.sync_copy(dst_hbm.at[wid], idx_d)
        plsc.subcore_barrier()

        @pl.loop(0, K)
        def _(j):
            pltpu.sync_copy(table_hbm.at[idx_s.at[j]], rows)
            pltpu.sync_copy(rows, acc.at[idx_d.at[j]], add=True)

        plsc.subcore_barrier()
        pltpu.sync_copy(acc.at[pl.ds(base, RPS)],
                        out_hbm.at[c].at[pl.ds(base, RPS)])

    return segsum


_sc_segsum_l1 = _make_sc_segsum(W1A)
_sc_segsum_l2 = _make_sc_segsum(W2A)


def _tc1_body(x_ref, wl_ref, wr_ref, yaug_ref, xr_ref):
    x = x_ref[...]
    y = jnp.dot(x, wl_ref[...], preferred_element_type=jnp.float32)
    ones = jnp.ones((NN, 1), jnp.float32)
    zeros = jnp.zeros((NN, W1A - 65), jnp.float32)
    yaug_ref[...] = jnp.concatenate([y, ones, zeros], axis=1)
    xr_ref[...] = jnp.dot(x, wr_ref[...], preferred_element_type=jnp.float32)


def _tc2_body(acc_ref, xr_ref, b1_ref, w2l_ref, w2r_ref,
              y2_ref, hr_ref, inv_ref):
    ssum = acc_ref[0, :NN, :] + acc_ref[1, :NN, :]
    feat = ssum[:, :64]
    deg = ssum[:, 64:65]
    inv = 1.0 / jnp.maximum(deg, 1.0)
    h = jnp.maximum(feat * inv + b1_ref[...] + xr_ref[...], 0.0)
    y2_ref[...] = jnp.dot(h, w2l_ref[...], preferred_element_type=jnp.float32)
    hr_ref[...] = jnp.dot(h, w2r_ref[...], preferred_element_type=jnp.float32)
    inv_ref[...] = jnp.broadcast_to(inv, (NN, W2A))


def _tc3_body(acc_ref, inv_ref, hr_ref, b2_ref, out_ref):
    ssum = acc_ref[0, :NN, :] + acc_ref[1, :NN, :]
    out_ref[...] = ssum * inv_ref[...] + hr_ref[...] + b2_ref[...]


def kernel(x, edge_index, W1l, b1, W1r, W2l, b2, W2r):
    src = edge_index[0].astype(jnp.int32)
    dst = edge_index[1].astype(jnp.int32)
    pad = NE_PAD - NE
    srcp = jnp.concatenate([src, jnp.zeros((pad,), jnp.int32)]
                           ).reshape(NW, K, CHUNK)
    dstp = jnp.concatenate([dst, jnp.full((pad,), NN, jnp.int32)]
                           ).reshape(NW, K, CHUNK)
    z1 = jnp.zeros((ACC_ROWS, W1A), jnp.float32)
    z2 = jnp.zeros((ACC_ROWS, W2A), jnp.float32)
    w2l_p = jnp.pad(W2l, ((0, 0), (0, W2A - 2)))
    w2r_p = jnp.pad(W2r, ((0, 0), (0, W2A - 2)))
    b1r = jnp.reshape(b1, (1, 64))
    b2r = jnp.reshape(jnp.pad(b2, (0, W2A - 2)), (1, W2A))

    y1aug, xr = pl.pallas_call(
        _tc1_body,
        out_shape=[jax.ShapeDtypeStruct((NN, W1A), jnp.float32),
                   jax.ShapeDtypeStruct((NN, 64), jnp.float32)],
    )(x, W1l, W1r)

    acc1 = _sc_segsum_l1(y1aug, srcp, dstp, z1)

    y2p, hr, inv = pl.pallas_call(
        _tc2_body,
        out_shape=[jax.ShapeDtypeStruct((NN, W2A), jnp.float32),
                   jax.ShapeDtypeStruct((NN, W2A), jnp.float32),
                   jax.ShapeDtypeStruct((NN, W2A), jnp.float32)],
    )(acc1, xr, b1r, w2l_p, w2r_p)

    acc2 = _sc_segsum_l2(y2p, srcp, dstp, z2)

    out16 = pl.pallas_call(
        _tc3_body,
        out_shape=jax.ShapeDtypeStruct((NN, W2A), jnp.float32),
    )(acc2, inv, hr, b2r)

    return out16[:, :2]


# speedup vs baseline: 6.8325x; 6.8325x over previous
import functools

import jax
import jax.numpy as jnp
from jax import lax
from jax.experimental import pallas as pl
from jax.experimental.pallas import tpu as pltpu
from jax.experimental.pallas import tpu_sc as plsc

NN = 10000
NE = 320000
NC = 2
NS = 16
NW = NC * NS
CHUNK = 128
K = 80
NE_PAD = NW * K * CHUNK
ACC_ROWS = 10240
RPS = ACC_ROWS // NS
W1A = 80
W2A = 16

_mesh = plsc.VectorSubcoreMesh(core_axis_name="c", subcore_axis_name="s")


def _make_sc_segsum(width):
    @functools.partial(
        pl.kernel,
        mesh=_mesh,
        compiler_params=pltpu.CompilerParams(use_tc_tiling_on_sc=False),
        out_type=jax.ShapeDtypeStruct((NC, ACC_ROWS, width), jnp.float32),
        scratch_types=[
            pltpu.VMEM((K, CHUNK), jnp.int32),
            pltpu.VMEM((K, CHUNK), jnp.int32),
            pltpu.VMEM((CHUNK, width), jnp.float32),
            pltpu.VMEM_SHARED((ACC_ROWS, width), jnp.float32),
        ],
    )
    def segsum(table_hbm, src_hbm, dst_hbm, zero_hbm, out_hbm,
               idx_s, idx_d, rows, acc):
        c = lax.axis_index("c")
        s = lax.axis_index("s")
        wid = c * NS + s
        base = s * RPS
        pltpu.sync_copy(zero_hbm.at[pl.ds(base, RPS)], acc.at[pl.ds(base, RPS)])
        pltpu.sync_copy(src_hbm.at[wid], idx_s)
        pltpu.sync_copy(dst_hbm.at[wid], idx_d)
        plsc.subcore_barrier()

        @pl.loop(0, K)
        def _(j):
            pltpu.sync_copy(table_hbm.at[idx_s.at[j]], rows)
            pltpu.sync_copy(rows, acc.at[idx_d.at[j]], add=True)

        plsc.subcore_barrier()
        pltpu.sync_copy(acc.at[pl.ds(base, RPS)],
                        out_hbm.at[c].at[pl.ds(base, RPS)])

    return segsum


_sc_segsum_l1 = _make_sc_segsum(W1A)
_sc_segsum_l2 = _make_sc_segsum(W2A)


def _tc1_body(x_ref, wl_ref, wr_ref, yaug_ref, xr_ref):
    x = x_ref[...]
    y = jnp.dot(x, wl_ref[...], preferred_element_type=jnp.float32)
    ones = jnp.ones((NN, 1), jnp.float32)
    zeros = jnp.zeros((NN, W1A - 65), jnp.float32)
    yaug_ref[...] = jnp.concatenate([y, ones, zeros], axis=1)
    xr_ref[...] = jnp.dot(x, wr_ref[...], preferred_element_type=jnp.float32)


def _tc2_body(acc_ref, xr_ref, b1_ref, w2l_ref, w2r_ref,
              y2_ref, hr_ref, inv_ref):
    ssum = acc_ref[0, :NN, :] + acc_ref[1, :NN, :]
    feat = ssum[:, :64]
    deg = ssum[:, 64:65]
    inv = 1.0 / jnp.maximum(deg, 1.0)
    h = jnp.maximum(feat * inv + b1_ref[...] + xr_ref[...], 0.0)
    y2_ref[...] = jnp.dot(h, w2l_ref[...], preferred_element_type=jnp.float32)
    hr_ref[...] = jnp.dot(h, w2r_ref[...], preferred_element_type=jnp.float32)
    inv_ref[...] = jnp.broadcast_to(inv, (NN, W2A))


def _tc3_body(acc_ref, inv_ref, hr_ref, b2_ref, out_ref):
    ssum = acc_ref[0, :NN, :] + acc_ref[1, :NN, :]
    out_ref[...] = ssum * inv_ref[...] + hr_ref[...] + b2_ref[...]


def kernel(x, edge_index, W1l, b1, W1r, W2l, b2, W2r):
    src = edge_index[0].astype(jnp.int32)
    dst = edge_index[1].astype(jnp.int32)
    pad = NE_PAD - NE
    srcp = jnp.concatenate([src, jnp.zeros((pad,), jnp.int32)]
                           ).reshape(NW, K, CHUNK)
    dstp = jnp.concatenate([dst, jnp.full((pad,), NN, jnp.int32)]
                           ).reshape(NW, K, CHUNK)
    z1 = jnp.zeros((ACC_ROWS, W1A), jnp.float32)
    z2 = jnp.zeros((ACC_ROWS, W2A), jnp.float32)
    w2l_p = jnp.pad(W2l, ((0, 0), (0, W2A - 2)))
    w2r_p = jnp.pad(W2r, ((0, 0), (0, W2A - 2)))
    b1r = jnp.reshape(b1, (1, 64))
    b2r = jnp.reshape(jnp.pad(b2, (0, W2A - 2)), (1, W2A))

    y1aug, xr = pl.pallas_call(
        _tc1_body,
        out_shape=[jax.ShapeDtypeStruct((NN, W1A), jnp.float32),
                   jax.ShapeDtypeStruct((NN, 64), jnp.float32)],
    )(x, W1l, W1r)

    acc1 = _sc_segsum_l1(y1aug, srcp, dstp, z1)

    y2p, hr, inv = pl.pallas_call(
        _tc2_body,
        out_shape=[jax.ShapeDtypeStruct((NN, W2A), jnp.float32),
                   jax.ShapeDtypeStruct((NN, W2A), jnp.float32),
                   jax.ShapeDtypeStruct((NN, W2A), jnp.float32)],
    )(acc1, xr, b1r, w2l_p, w2r_p)

    acc2 = _sc_segsum_l2(y2p, srcp, dstp, z2)

    out16 = pl.pallas_call(
        _tc3_body,
        out_shape=jax.ShapeDtypeStruct((NN, W2A), jnp.float32),
    )(acc2, inv, hr, b2r)

    return out16[:, :2]


# speedup vs baseline: 7.6282x; 1.1165x over previous
import functools

import jax
import jax.numpy as jnp
from jax import lax
from jax.experimental import pallas as pl
from jax.experimental.pallas import tpu as pltpu
from jax.experimental.pallas import tpu_sc as plsc

NN = 10000
NE = 320000
NC = 2
NS = 16
NW = NC * NS
CHUNK = 512
K = 20
NE_PAD = NW * K * CHUNK
ACC_ROWS = 10240
RPS = ACC_ROWS // NS
W1A = 80
W2A = 16

_mesh = plsc.VectorSubcoreMesh(core_axis_name="c", subcore_axis_name="s")


def _make_sc_segsum(width):
    @functools.partial(
        pl.kernel,
        mesh=_mesh,
        compiler_params=pltpu.CompilerParams(use_tc_tiling_on_sc=False),
        out_type=jax.ShapeDtypeStruct((NC, ACC_ROWS, width), jnp.float32),
        scratch_types=[
            pltpu.VMEM((K, CHUNK), jnp.int32),
            pltpu.VMEM((K, CHUNK), jnp.int32),
            pltpu.VMEM((CHUNK, width), jnp.float32),
            pltpu.VMEM_SHARED((ACC_ROWS, width), jnp.float32),
        ],
    )
    def segsum(table_hbm, src_hbm, dst_hbm, zero_hbm, out_hbm,
               idx_s, idx_d, rows, acc):
        c = lax.axis_index("c")
        s = lax.axis_index("s")
        wid = c * NS + s
        base = s * RPS
        pltpu.sync_copy(zero_hbm.at[pl.ds(base, RPS)], acc.at[pl.ds(base, RPS)])
        pltpu.sync_copy(src_hbm.at[wid], idx_s)
        pltpu.sync_copy(dst_hbm.at[wid], idx_d)
        plsc.subcore_barrier()

        @pl.loop(0, K)
        def _(j):
            pltpu.sync_copy(table_hbm.at[idx_s.at[j]], rows)
            pltpu.sync_copy(rows, acc.at[idx_d.at[j]], add=True)

        plsc.subcore_barrier()
        pltpu.sync_copy(acc.at[pl.ds(base, RPS)],
                        out_hbm.at[c].at[pl.ds(base, RPS)])

    return segsum


_sc_segsum_l1 = _make_sc_segsum(W1A)
_sc_segsum_l2 = _make_sc_segsum(W2A)


def _tc1_body(x_ref, wl_ref, wr_ref, yaug_ref, xr_ref):
    x = x_ref[...]
    y = jnp.dot(x, wl_ref[...], preferred_element_type=jnp.float32)
    ones = jnp.ones((NN, 1), jnp.float32)
    zeros = jnp.zeros((NN, W1A - 65), jnp.float32)
    yaug_ref[...] = jnp.concatenate([y, ones, zeros], axis=1)
    xr_ref[...] = jnp.dot(x, wr_ref[...], preferred_element_type=jnp.float32)


def _tc2_body(acc_ref, xr_ref, b1_ref, w2l_ref, w2r_ref,
              y2_ref, hr_ref, inv_ref):
    ssum = acc_ref[0, :NN, :] + acc_ref[1, :NN, :]
    feat = ssum[:, :64]
    deg = ssum[:, 64:65]
    inv = 1.0 / jnp.maximum(deg, 1.0)
    h = jnp.maximum(feat * inv + b1_ref[...] + xr_ref[...], 0.0)
    y2_ref[...] = jnp.dot(h, w2l_ref[...], preferred_element_type=jnp.float32)
    hr_ref[...] = jnp.dot(h, w2r_ref[...], preferred_element_type=jnp.float32)
    inv_ref[...] = jnp.broadcast_to(inv, (NN, W2A))


def _tc3_body(acc_ref, inv_ref, hr_ref, b2_ref, out_ref):
    ssum = acc_ref[0, :NN, :] + acc_ref[1, :NN, :]
    out_ref[...] = ssum * inv_ref[...] + hr_ref[...] + b2_ref[...]


def kernel(x, edge_index, W1l, b1, W1r, W2l, b2, W2r):
    src = edge_index[0].astype(jnp.int32)
    dst = edge_index[1].astype(jnp.int32)
    pad = NE_PAD - NE
    srcp = jnp.concatenate([src, jnp.zeros((pad,), jnp.int32)]
                           ).reshape(NW, K, CHUNK)
    dstp = jnp.concatenate([dst, jnp.full((pad,), NN, jnp.int32)]
                           ).reshape(NW, K, CHUNK)
    z1 = jnp.zeros((ACC_ROWS, W1A), jnp.float32)
    z2 = jnp.zeros((ACC_ROWS, W2A), jnp.float32)
    w2l_p = jnp.pad(W2l, ((0, 0), (0, W2A - 2)))
    w2r_p = jnp.pad(W2r, ((0, 0), (0, W2A - 2)))
    b1r = jnp.reshape(b1, (1, 64))
    b2r = jnp.reshape(jnp.pad(b2, (0, W2A - 2)), (1, W2A))

    y1aug, xr = pl.pallas_call(
        _tc1_body,
        out_shape=[jax.ShapeDtypeStruct((NN, W1A), jnp.float32),
                   jax.ShapeDtypeStruct((NN, 64), jnp.float32)],
    )(x, W1l, W1r)

    acc1 = _sc_segsum_l1(y1aug, srcp, dstp, z1)

    y2p, hr, inv = pl.pallas_call(
        _tc2_body,
        out_shape=[jax.ShapeDtypeStruct((NN, W2A), jnp.float32),
                   jax.ShapeDtypeStruct((NN, W2A), jnp.float32),
                   jax.ShapeDtypeStruct((NN, W2A), jnp.float32)],
    )(acc1, xr, b1r, w2l_p, w2r_p)

    acc2 = _sc_segsum_l2(y2p, srcp, dstp, z2)

    out16 = pl.pallas_call(
        _tc3_body,
        out_shape=jax.ShapeDtypeStruct((NN, W2A), jnp.float32),
    )(acc2, inv, hr, b2r)

    return out16[:, :2]


# speedup vs baseline: 8.6478x; 1.1337x over previous
import functools

import jax
import jax.numpy as jnp
from jax import lax
from jax.experimental import pallas as pl
from jax.experimental.pallas import tpu as pltpu
from jax.experimental.pallas import tpu_sc as plsc

NN = 10000
NE = 320000
NC = 2
NS = 16
NW = NC * NS
CHUNK = 256
K = 40
NE_PAD = NW * K * CHUNK
ACC_ROWS = 10240
RPS = ACC_ROWS // NS
W1A = 80
W2A = 16

_mesh = plsc.VectorSubcoreMesh(core_axis_name="c", subcore_axis_name="s")


def _make_sc_segsum(width):
    @functools.partial(
        pl.kernel,
        mesh=_mesh,
        compiler_params=pltpu.CompilerParams(use_tc_tiling_on_sc=False),
        out_type=jax.ShapeDtypeStruct((NC, ACC_ROWS, width), jnp.float32),
        scratch_types=[
            pltpu.VMEM((K, CHUNK), jnp.int32),
            pltpu.VMEM((K, CHUNK), jnp.int32),
            pltpu.VMEM((CHUNK, width), jnp.float32),
            pltpu.VMEM((CHUNK, width), jnp.float32),
            pltpu.VMEM_SHARED((ACC_ROWS, width), jnp.float32),
            pltpu.SemaphoreType.DMA,
            pltpu.SemaphoreType.DMA,
        ],
    )
    def segsum(table_hbm, src_hbm, dst_hbm, zero_hbm, out_hbm,
               idx_s, idx_d, rows0, rows1, acc, gsem0, gsem1):
        c = lax.axis_index("c")
        s = lax.axis_index("s")
        wid = c * NS + s
        base = s * RPS
        pltpu.sync_copy(zero_hbm.at[pl.ds(base, RPS)], acc.at[pl.ds(base, RPS)])
        pltpu.sync_copy(src_hbm.at[wid], idx_s)
        pltpu.sync_copy(dst_hbm.at[wid], idx_d)
        plsc.subcore_barrier()

        pltpu.async_copy(table_hbm.at[idx_s.at[0]], rows0, gsem0)

        @pl.loop(0, K, step=2)
        def _(j):
            pltpu.async_copy(table_hbm.at[idx_s.at[j + 1]], rows1, gsem1)
            pltpu.make_async_copy(table_hbm.at[idx_s.at[j]], rows0, gsem0
                                  ).wait()
            pltpu.sync_copy(rows0, acc.at[idx_d.at[j]], add=True)

            @pl.when(j + 2 < K)
            def _():
                pltpu.async_copy(table_hbm.at[idx_s.at[j + 2]], rows0, gsem0)

            pltpu.make_async_copy(table_hbm.at[idx_s.at[j + 1]], rows1, gsem1
                                  ).wait()
            pltpu.sync_copy(rows1, acc.at[idx_d.at[j + 1]], add=True)

        plsc.subcore_barrier()
        pltpu.sync_copy(acc.at[pl.ds(base, RPS)],
                        out_hbm.at[c].at[pl.ds(base, RPS)])

    return segsum


_sc_segsum_l1 = _make_sc_segsum(W1A)
_sc_segsum_l2 = _make_sc_segsum(W2A)


def _tc1_body(x_ref, wl_ref, wr_ref, yaug_ref, xr_ref):
    x = x_ref[...]
    y = jnp.dot(x, wl_ref[...], preferred_element_type=jnp.float32)
    ones = jnp.ones((NN, 1), jnp.float32)
    zeros = jnp.zeros((NN, W1A - 65), jnp.float32)
    yaug_ref[...] = jnp.concatenate([y, ones, zeros], axis=1)
    xr_ref[...] = jnp.dot(x, wr_ref[...], preferred_element_type=jnp.float32)


def _tc2_body(acc_ref, xr_ref, b1_ref, w2l_ref, w2r_ref,
              y2_ref, hr_ref, inv_ref):
    ssum = acc_ref[0, :NN, :] + acc_ref[1, :NN, :]
    feat = ssum[:, :64]
    deg = ssum[:, 64:65]
    inv = 1.0 / jnp.maximum(deg, 1.0)
    h = jnp.maximum(feat * inv + b1_ref[...] + xr_ref[...], 0.0)
    y2_ref[...] = jnp.dot(h, w2l_ref[...], preferred_element_type=jnp.float32)
    hr_ref[...] = jnp.dot(h, w2r_ref[...], preferred_element_type=jnp.float32)
    inv_ref[...] = jnp.broadcast_to(inv, (NN, W2A))


def _tc3_body(acc_ref, inv_ref, hr_ref, b2_ref, out_ref):
    ssum = acc_ref[0, :NN, :] + acc_ref[1, :NN, :]
    out_ref[...] = ssum * inv_ref[...] + hr_ref[...] + b2_ref[...]


def kernel(x, edge_index, W1l, b1, W1r, W2l, b2, W2r):
    src = edge_index[0].astype(jnp.int32)
    dst = edge_index[1].astype(jnp.int32)
    pad = NE_PAD - NE
    srcp = jnp.concatenate([src, jnp.zeros((pad,), jnp.int32)]
                           ).reshape(NW, K, CHUNK)
    dstp = jnp.concatenate([dst, jnp.full((pad,), NN, jnp.int32)]
                           ).reshape(NW, K, CHUNK)
    z1 = jnp.zeros((ACC_ROWS, W1A), jnp.float32)
    z2 = jnp.zeros((ACC_ROWS, W2A), jnp.float32)
    w2l_p = jnp.pad(W2l, ((0, 0), (0, W2A - 2)))
    w2r_p = jnp.pad(W2r, ((0, 0), (0, W2A - 2)))
    b1r = jnp.reshape(b1, (1, 64))
    b2r = jnp.reshape(jnp.pad(b2, (0, W2A - 2)), (1, W2A))

    y1aug, xr = pl.pallas_call(
        _tc1_body,
        out_shape=[jax.ShapeDtypeStruct((NN, W1A), jnp.float32),
                   jax.ShapeDtypeStruct((NN, 64), jnp.float32)],
    )(x, W1l, W1r)

    acc1 = _sc_segsum_l1(y1aug, srcp, dstp, z1)

    y2p, hr, inv = pl.pallas_call(
        _tc2_body,
        out_shape=[jax.ShapeDtypeStruct((NN, W2A), jnp.float32),
                   jax.ShapeDtypeStruct((NN, W2A), jnp.float32),
                   jax.ShapeDtypeStruct((NN, W2A), jnp.float32)],
    )(acc1, xr, b1r, w2l_p, w2r_p)

    acc2 = _sc_segsum_l2(y2p, srcp, dstp, z2)

    out16 = pl.pallas_call(
        _tc3_body,
        out_shape=jax.ShapeDtypeStruct((NN, W2A), jnp.float32),
    )(acc2, inv, hr, b2r)

    return out16[:, :2]


# speedup vs baseline: 8.9825x; 1.0387x over previous
import functools

import jax
import jax.numpy as jnp
from jax import lax
from jax.experimental import pallas as pl
from jax.experimental.pallas import tpu as pltpu
from jax.experimental.pallas import tpu_sc as plsc

NN = 10000
NE = 320000
NC = 2
NS = 16
NW = NC * NS
CHUNK = 256
NCHUNKS = 1280
NE_PAD = NCHUNKS * CHUNK
K1_0, K1_1 = 64, 16
K2_0, K2_1 = 54, 26
ACC_ROWS = 10240
RPS = ACC_ROWS // NS
W1A = 80
W2A = 16

_mesh = plsc.VectorSubcoreMesh(core_axis_name="c", subcore_axis_name="s")


def _make_sc_segsum(width, k0, k1):
    kmax = max(k0, k1)

    @functools.partial(
        pl.kernel,
        mesh=_mesh,
        compiler_params=pltpu.CompilerParams(use_tc_tiling_on_sc=False),
        out_type=jax.ShapeDtypeStruct((NC, ACC_ROWS, width), jnp.float32),
        scratch_types=[
            pltpu.VMEM((kmax, CHUNK), jnp.int32),
            pltpu.VMEM((kmax, CHUNK), jnp.int32),
            pltpu.VMEM((CHUNK, width), jnp.float32),
            pltpu.VMEM((CHUNK, width), jnp.float32),
            pltpu.VMEM_SHARED((ACC_ROWS, width), jnp.float32),
            pltpu.SemaphoreType.DMA,
            pltpu.SemaphoreType.DMA,
        ],
    )
    def segsum(table_hbm, src_hbm, dst_hbm, zero_hbm, out_hbm,
               idx_s, idx_d, rows0, rows1, acc, gsem0, gsem1):
        c = lax.axis_index("c")
        s = lax.axis_index("s")
        base = s * RPS
        pltpu.sync_copy(zero_hbm.at[pl.ds(base, RPS)], acc.at[pl.ds(base, RPS)])
        plsc.subcore_barrier()

        def run(start, k):
            pltpu.sync_copy(src_hbm.at[pl.ds(start, k)],
                            idx_s.at[pl.ds(0, k)])
            pltpu.sync_copy(dst_hbm.at[pl.ds(start, k)],
                            idx_d.at[pl.ds(0, k)])
            pltpu.async_copy(table_hbm.at[idx_s.at[0]], rows0, gsem0)

            @pl.loop(0, k, step=2)
            def _(j):
                pltpu.async_copy(table_hbm.at[idx_s.at[j + 1]], rows1, gsem1)
                pltpu.make_async_copy(table_hbm.at[idx_s.at[j]], rows0, gsem0
                                      ).wait()
                pltpu.sync_copy(rows0, acc.at[idx_d.at[j]], add=True)

                @pl.when(j + 2 < k)
                def _():
                    pltpu.async_copy(table_hbm.at[idx_s.at[j + 2]], rows0,
                                     gsem0)

                pltpu.make_async_copy(table_hbm.at[idx_s.at[j + 1]], rows1,
                                      gsem1).wait()
                pltpu.sync_copy(rows1, acc.at[idx_d.at[j + 1]], add=True)

        @pl.when(c == 0)
        def _():
            run(s * k0, k0)

        @pl.when(c == 1)
        def _():
            run(NS * k0 + s * k1, k1)

        plsc.subcore_barrier()
        pltpu.sync_copy(acc.at[pl.ds(base, RPS)],
                        out_hbm.at[c].at[pl.ds(base, RPS)])

    return segsum


_sc_segsum_l1 = _make_sc_segsum(W1A, K1_0, K1_1)
_sc_segsum_l2 = _make_sc_segsum(W2A, K2_0, K2_1)


def _tc1_body(x_ref, wl_ref, wr_ref, yaug_ref, xr_ref):
    x = x_ref[...]
    y = jnp.dot(x, wl_ref[...], preferred_element_type=jnp.float32)
    ones = jnp.ones((NN, 1), jnp.float32)
    zeros = jnp.zeros((NN, W1A - 65), jnp.float32)
    yaug_ref[...] = jnp.concatenate([y, ones, zeros], axis=1)
    xr_ref[...] = jnp.dot(x, wr_ref[...], preferred_element_type=jnp.float32)


def _tc2_body(acc_ref, xr_ref, b1_ref, w2l_ref, w2r_ref,
              y2_ref, hr_ref, inv_ref):
    ssum = acc_ref[0, :NN, :] + acc_ref[1, :NN, :]
    feat = ssum[:, :64]
    deg = ssum[:, 64:65]
    inv = 1.0 / jnp.maximum(deg, 1.0)
    h = jnp.maximum(feat * inv + b1_ref[...] + xr_ref[...], 0.0)
    y2_ref[...] = jnp.dot(h, w2l_ref[...], preferred_element_type=jnp.float32)
    hr_ref[...] = jnp.dot(h, w2r_ref[...], preferred_element_type=jnp.float32)
    inv_ref[...] = jnp.broadcast_to(inv, (NN, W2A))


def _tc3_body(acc_ref, inv_ref, hr_ref, b2_ref, out_ref):
    ssum = acc_ref[0, :NN, :] + acc_ref[1, :NN, :]
    out_ref[...] = ssum * inv_ref[...] + hr_ref[...] + b2_ref[...]


def kernel(x, edge_index, W1l, b1, W1r, W2l, b2, W2r):
    src = edge_index[0].astype(jnp.int32)
    dst = edge_index[1].astype(jnp.int32)
    pad = NE_PAD - NE
    srcp = jnp.concatenate([src, jnp.zeros((pad,), jnp.int32)]
                           ).reshape(NCHUNKS, CHUNK)
    trash = NN + jnp.arange(pad, dtype=jnp.int32) % (ACC_ROWS - NN)
    dstp = jnp.concatenate([dst, trash]).reshape(NCHUNKS, CHUNK)
    z1 = jnp.zeros((ACC_ROWS, W1A), jnp.float32)
    z2 = jnp.zeros((ACC_ROWS, W2A), jnp.float32)
    w2l_p = jnp.pad(W2l, ((0, 0), (0, W2A - 2)))
    w2r_p = jnp.pad(W2r, ((0, 0), (0, W2A - 2)))
    b1r = jnp.reshape(b1, (1, 64))
    b2r = jnp.reshape(jnp.pad(b2, (0, W2A - 2)), (1, W2A))

    y1aug, xr = pl.pallas_call(
        _tc1_body,
        out_shape=[jax.ShapeDtypeStruct((NN, W1A), jnp.float32),
                   jax.ShapeDtypeStruct((NN, 64), jnp.float32)],
    )(x, W1l, W1r)

    acc1 = _sc_segsum_l1(y1aug, srcp, dstp, z1)

    y2p, hr, inv = pl.pallas_call(
        _tc2_body,
        out_shape=[jax.ShapeDtypeStruct((NN, W2A), jnp.float32),
                   jax.ShapeDtypeStruct((NN, W2A), jnp.float32),
                   jax.ShapeDtypeStruct((NN, W2A), jnp.float32)],
    )(acc1, xr, b1r, w2l_p, w2r_p)

    acc2 = _sc_segsum_l2(y2p, srcp, dstp, z2)

    out16 = pl.pallas_call(
        _tc3_body,
        out_shape=jax.ShapeDtypeStruct((NN, W2A), jnp.float32),
    )(acc2, inv, hr, b2r)

    return out16[:, :2]
